# Initial kernel scaffold; baseline (speedup 1.0000x reference)
import jax
import jax.numpy as jnp
from jax.experimental import pallas as pl


def kernel(input_ids, image_features, image_token_id, W):
    raise NotImplementedError("write your pallas kernel here")


# speedup vs baseline: 3.5143x; 3.5143x over previous
import functools

import jax
import jax.numpy as jnp
from jax import lax
from jax.experimental import pallas as pl
from jax.experimental.pallas import tpu as pltpu
from jax.experimental.pallas import tpu_sc as plsc


def _build_sc_kernel(B, S, P, H, V):
    info = plsc.get_sparse_core_info()
    NC, NS, L = info.num_cores, info.num_subcores, info.num_lanes
    NW = NC * NS
    assert (B * S) % NW == 0
    TPW = (B * S) // NW
    WPR = NW // B
    assert S % WPR == 0 and TPW == S // WPR
    CH = 32
    NCHUNK = TPW // CH
    IPW = P // WPR

    mesh = plsc.VectorSubcoreMesh(core_axis_name="c", subcore_axis_name="s")

    @functools.partial(
        pl.kernel,
        out_type=[
            jax.ShapeDtypeStruct((B * S, H), jnp.float32),
            jax.ShapeDtypeStruct((B * S,), jnp.int32),
        ],
        mesh=mesh,
        scratch_types=[
            pltpu.VMEM((S,), jnp.int32),
            pltpu.VMEM((L,), jnp.int32),
            pltpu.VMEM((TPW,), jnp.int32),
            pltpu.VMEM((CH, H), jnp.float32),
            pltpu.VMEM((CH, H), jnp.float32),
            pltpu.VMEM((L,), jnp.int32),
            pltpu.SemaphoreType.DMA,
        ],
    )
    def body(ids_hbm, img_hbm, tid_hbm, w_hbm, out_hbm, mask_hbm,
             row_v, tid_v, mask_v, buf0, buf1, sidx_v, sem):
        c = lax.axis_index("c")
        s = lax.axis_index("s")
        wid = c * NS + s
        b = wid // WPR
        kw = wid % WPR
        loc = kw * TPW
        t0 = b * S + loc

        pltpu.sync_copy(ids_hbm.at[pl.ds(b * S, S)], row_v)
        pltpu.sync_copy(tid_hbm, tid_v)
        tidv = tid_v[...]

        def scan_body(i, acc):
            v = row_v[pl.ds(i * L, L)]
            posv = lax.iota(jnp.int32, L) + i * L
            return jnp.minimum(acc, jnp.where(v == tidv, posv, S))

        fpv = lax.fori_loop(0, S // L, scan_body,
                            jnp.full((L,), S, jnp.int32))
        dnums = lax.GatherDimensionNumbers(
            offset_dims=(), collapsed_slice_dims=(0,), start_index_map=(0,))
        for st in (1, 2, 4, 8):
            perm = (lax.iota(jnp.int32, L) ^ st)[:, None]
            fpv = jnp.minimum(
                fpv, lax.gather(fpv, perm, dnums, (1,),
                                mode=lax.GatherScatterMode.PROMISE_IN_BOUNDS))
        fp = jnp.squeeze(lax.slice(fpv, (0,), (1,)))
        valid = fp <= S - P

        for k in range(NCHUNK):
            buf = buf0 if k % 2 == 0 else buf1
            idx = row_v.at[pl.ds(loc + k * CH, CH)]
            pltpu.async_copy(w_hbm.at[idx], buf, sem).wait()
            pltpu.sync_copy(buf, out_hbm.at[pl.ds(t0 + k * CH, CH)])

        iota = lax.iota(jnp.int32, L)
        one = jnp.full((L,), 1, jnp.int32)
        zero = jnp.full((L,), 0, jnp.int32)
        for j in range(TPW // L):
            v = row_v[pl.ds(loc + j * L, L)]
            mask_v[pl.ds(j * L, L)] = jnp.where(v != -100, one, zero)
        pltpu.sync_copy(mask_v, mask_hbm.at[pl.ds(t0, TPW)])

        plsc.subcore_barrier()

        @pl.when(valid)
        def _image_overwrite():
            pltpu.sync_copy(img_hbm.at[pl.ds(b * P + kw * IPW, IPW)], buf0)
            for h in range(IPW // L):
                sidx_v[...] = b * S + fpv + kw * IPW + h * L + iota
                pltpu.async_copy(buf0.at[pl.ds(h * L, L)],
                                 out_hbm.at[sidx_v], sem).wait()

    return body


def kernel(input_ids, image_features, image_token_id, W):
    B, S = input_ids.shape
    _, P, H = image_features.shape
    V = W.shape[0]

    ids = input_ids.astype(jnp.int32).reshape(B * S)
    tid = jnp.full((16,), image_token_id, dtype=jnp.int32)
    img2 = image_features.reshape(B * P, H)

    sc = _build_sc_kernel(B, S, P, H, V)
    out, mask = sc(ids, img2, tid, W)
    return out.reshape(B, S, H), mask.reshape(B, S)


# speedup vs baseline: 3.8980x; 1.1092x over previous
import functools

import jax
import jax.numpy as jnp
from jax import lax
from jax.experimental import pallas as pl
from jax.experimental.pallas import tpu as pltpu
from jax.experimental.pallas import tpu_sc as plsc


def _build_sc_kernel(B, S, P, H, V):
    info = plsc.get_sparse_core_info()
    NC, NS, L = info.num_cores, info.num_subcores, info.num_lanes
    NW = NC * NS
    assert (B * S) % NW == 0
    TPW = (B * S) // NW
    WPR = NW // B
    assert S % WPR == 0 and TPW == S // WPR
    CH = 32
    NCHUNK = TPW // CH
    IPW = P // WPR

    mesh = plsc.VectorSubcoreMesh(core_axis_name="c", subcore_axis_name="s")

    @functools.partial(
        pl.kernel,
        out_type=[
            jax.ShapeDtypeStruct((B * S, H), jnp.float32),
            jax.ShapeDtypeStruct((B * S,), jnp.int32),
        ],
        mesh=mesh,
        scratch_types=[
            pltpu.VMEM((S,), jnp.int32),
            pltpu.VMEM((L,), jnp.int32),
            pltpu.VMEM((TPW,), jnp.int32),
            pltpu.VMEM((CH, H), jnp.float32),
            pltpu.VMEM((CH, H), jnp.float32),
            pltpu.VMEM((CH, H), jnp.float32),
            pltpu.VMEM((L,), jnp.int32),
            pltpu.SemaphoreType.DMA,
            pltpu.SemaphoreType.DMA,
            pltpu.SemaphoreType.DMA,
            pltpu.SemaphoreType.DMA,
            pltpu.SemaphoreType.DMA,
            pltpu.SemaphoreType.DMA,
            pltpu.SemaphoreType.DMA,
        ],
    )
    def body(ids_hbm, img_hbm, tid_hbm, w_hbm, out_hbm, mask_hbm,
             row_v, tid_v, mask_v, buf0, buf1, buf2, sidx_v,
             gs0, gs1, gs2, os0, os1, os2, sem):
        bufs = (buf0, buf1, buf2)
        gsems = (gs0, gs1, gs2)
        osems = (os0, os1, os2)
        NBUF = 3
        c = lax.axis_index("c")
        s = lax.axis_index("s")
        wid = c * NS + s
        b = wid // WPR
        kw = wid % WPR
        loc = kw * TPW
        t0 = b * S + loc

        pltpu.sync_copy(ids_hbm.at[pl.ds(b * S, S)], row_v)
        pltpu.sync_copy(tid_hbm, tid_v)
        tidv = tid_v[...]

        gd = [None] * NBUF
        od = [None] * NBUF

        def fire_g(k):
            gd[k % NBUF] = pltpu.async_copy(
                w_hbm.at[row_v.at[pl.ds(loc + k * CH, CH)]],
                bufs[k % NBUF], gsems[k % NBUF])

        def fire_o(k):
            od[k % NBUF] = pltpu.async_copy(
                bufs[k % NBUF], out_hbm.at[pl.ds(t0 + k * CH, CH)],
                osems[k % NBUF])

        fire_g(0)
        fire_g(1)

        def scan_body(i, acc):
            v = row_v[pl.ds(i * L, L)]
            posv = lax.iota(jnp.int32, L) + i * L
            return jnp.minimum(acc, jnp.where(v == tidv, posv, S))

        fpv = lax.fori_loop(0, S // L, scan_body,
                            jnp.full((L,), S, jnp.int32))
        dnums = lax.GatherDimensionNumbers(
            offset_dims=(), collapsed_slice_dims=(0,), start_index_map=(0,))
        for st in (1, 2, 4, 8):
            perm = (lax.iota(jnp.int32, L) ^ st)[:, None]
            fpv = jnp.minimum(
                fpv, lax.gather(fpv, perm, dnums, (1,),
                                mode=lax.GatherScatterMode.PROMISE_IN_BOUNDS))
        fp = jnp.squeeze(lax.slice(fpv, (0,), (1,)))
        valid = fp <= S - P

        iota = lax.iota(jnp.int32, L)
        one = jnp.full((L,), 1, jnp.int32)
        zero = jnp.full((L,), 0, jnp.int32)
        for j in range(TPW // L):
            v = row_v[pl.ds(loc + j * L, L)]
            mask_v[pl.ds(j * L, L)] = jnp.where(v != -100, one, zero)
        pltpu.sync_copy(mask_v, mask_hbm.at[pl.ds(t0, TPW)])

        o_pending = [False] * NCHUNK
        for k in range(NCHUNK):
            gd[k % NBUF].wait()
            fire_o(k)
            o_pending[k] = True
            if k + 2 < NCHUNK:
                j = k - 1
                if j >= 0:
                    od[j % NBUF].wait()
                    o_pending[j] = False
                fire_g(k + 2)
        for k in range(NCHUNK):
            if o_pending[k]:
                od[k % NBUF].wait()

        plsc.subcore_barrier()

        @pl.when(valid)
        def _image_overwrite():
            pltpu.sync_copy(img_hbm.at[pl.ds(b * P + kw * IPW, IPW)], buf0)
            for h in range(IPW // L):
                sidx_v[...] = b * S + fpv + kw * IPW + h * L + iota
                pltpu.async_copy(buf0.at[pl.ds(h * L, L)],
                                 out_hbm.at[sidx_v], sem).wait()

    return body


def kernel(input_ids, image_features, image_token_id, W):
    B, S = input_ids.shape
    _, P, H = image_features.shape
    V = W.shape[0]

    ids = input_ids.astype(jnp.int32).reshape(B * S)
    tid = jnp.full((16,), image_token_id, dtype=jnp.int32)
    img2 = image_features.reshape(B * P, H)

    sc = _build_sc_kernel(B, S, P, H, V)
    out, mask = sc(ids, img2, tid, W)
    return out.reshape(B, S, H), mask.reshape(B, S)


# speedup vs baseline: 4.0305x; 1.0340x over previous
import functools

import jax
import jax.numpy as jnp
from jax import lax
from jax.experimental import pallas as pl
from jax.experimental.pallas import tpu as pltpu
from jax.experimental.pallas import tpu_sc as plsc


def _build_sc_kernel(B, S, P, H, V):
    info = plsc.get_sparse_core_info()
    NC, NS, L = info.num_cores, info.num_subcores, info.num_lanes
    NW = NC * NS
    assert (B * S) % NW == 0
    TPW = (B * S) // NW
    WPR = NW // B
    assert S % WPR == 0 and TPW == S // WPR
    CH = 32
    NCHUNK = TPW // CH
    IPW = P // WPR

    mesh = plsc.VectorSubcoreMesh(core_axis_name="c", subcore_axis_name="s")

    @functools.partial(
        pl.kernel,
        out_type=[
            jax.ShapeDtypeStruct((B, S, H), jnp.float32),
            jax.ShapeDtypeStruct((B, S), jnp.int32),
        ],
        mesh=mesh,
        scratch_types=[
            pltpu.VMEM((S,), jnp.int32),
            pltpu.VMEM((L,), jnp.int32),
            pltpu.VMEM((TPW,), jnp.int32),
            pltpu.VMEM((CH, H), jnp.float32),
            pltpu.VMEM((CH, H), jnp.float32),
            pltpu.VMEM((CH, H), jnp.float32),
            pltpu.VMEM((L,), jnp.int32),
            pltpu.SemaphoreType.DMA,
            pltpu.SemaphoreType.DMA,
            pltpu.SemaphoreType.DMA,
            pltpu.SemaphoreType.DMA,
            pltpu.SemaphoreType.DMA,
            pltpu.SemaphoreType.DMA,
            pltpu.SemaphoreType.DMA,
        ],
    )
    def body(ids_hbm, img_hbm, tid_hbm, w_hbm, out_hbm, mask_hbm,
             row_v, tid_v, mask_v, buf0, buf1, buf2, sidx_v,
             gs0, gs1, gs2, os0, os1, os2, sem):
        bufs = (buf0, buf1, buf2)
        gsems = (gs0, gs1, gs2)
        osems = (os0, os1, os2)
        NBUF = 3
        c = lax.axis_index("c")
        s = lax.axis_index("s")
        wid = c * NS + s
        b = wid // WPR
        kw = wid % WPR
        loc = kw * TPW
        t0 = b * S + loc

        pltpu.sync_copy(ids_hbm.at[b], row_v)
        pltpu.sync_copy(tid_hbm, tid_v)
        tidv = tid_v[...]

        gd = [None] * NBUF
        od = [None] * NBUF

        def fire_g(k):
            gd[k % NBUF] = pltpu.async_copy(
                w_hbm.at[row_v.at[pl.ds(loc + k * CH, CH)]],
                bufs[k % NBUF], gsems[k % NBUF])

        def fire_o(k):
            od[k % NBUF] = pltpu.async_copy(
                bufs[k % NBUF], out_hbm.at[b, pl.ds(loc + k * CH, CH)],
                osems[k % NBUF])

        fire_g(0)
        fire_g(1)

        def scan_body(i, acc):
            v = row_v[pl.ds(i * L, L)]
            posv = lax.iota(jnp.int32, L) + i * L
            return jnp.minimum(acc, jnp.where(v == tidv, posv, S))

        fpv = lax.fori_loop(0, S // L, scan_body,
                            jnp.full((L,), S, jnp.int32))
        dnums = lax.GatherDimensionNumbers(
            offset_dims=(), collapsed_slice_dims=(0,), start_index_map=(0,))
        for st in (1, 2, 4, 8):
            perm = (lax.iota(jnp.int32, L) ^ st)[:, None]
            fpv = jnp.minimum(
                fpv, lax.gather(fpv, perm, dnums, (1,),
                                mode=lax.GatherScatterMode.PROMISE_IN_BOUNDS))
        fp = jnp.squeeze(lax.slice(fpv, (0,), (1,)))
        valid = fp <= S - P

        iota = lax.iota(jnp.int32, L)
        one = jnp.full((L,), 1, jnp.int32)
        zero = jnp.full((L,), 0, jnp.int32)
        for j in range(TPW // L):
            v = row_v[pl.ds(loc + j * L, L)]
            mask_v[pl.ds(j * L, L)] = jnp.where(v != -100, one, zero)
        pltpu.sync_copy(mask_v, mask_hbm.at[b, pl.ds(loc, TPW)])

        o_pending = [False] * NCHUNK
        for k in range(NCHUNK):
            gd[k % NBUF].wait()
            fire_o(k)
            o_pending[k] = True
            if k + 2 < NCHUNK:
                j = k - 1
                if j >= 0:
                    od[j % NBUF].wait()
                    o_pending[j] = False
                fire_g(k + 2)
        for k in range(NCHUNK):
            if o_pending[k]:
                od[k % NBUF].wait()

        plsc.subcore_barrier()

        @pl.when(valid)
        def _image_overwrite():
            pltpu.sync_copy(img_hbm.at[b, pl.ds(kw * IPW, IPW)], buf0)
            for h in range(IPW // L):
                sidx_v[...] = fpv + kw * IPW + h * L + iota
                pltpu.async_copy(buf0.at[pl.ds(h * L, L)],
                                 out_hbm.at[b].at[sidx_v], sem).wait()

    return body


def kernel(input_ids, image_features, image_token_id, W):
    B, S = input_ids.shape
    _, P, H = image_features.shape
    V = W.shape[0]

    ids = input_ids.astype(jnp.int32)
    tid = jnp.full((16,), image_token_id, dtype=jnp.int32)

    sc = _build_sc_kernel(B, S, P, H, V)
    out, mask = sc(ids, image_features, tid, W)
    return out, mask


# speedup vs baseline: 4.1165x; 1.0213x over previous
import functools

import jax
import jax.numpy as jnp
from jax import lax
from jax.experimental import pallas as pl
from jax.experimental.pallas import tpu as pltpu
from jax.experimental.pallas import tpu_sc as plsc


def _build_sc_kernel(B, S, P, H, V):
    info = plsc.get_sparse_core_info()
    NC, NS, L = info.num_cores, info.num_subcores, info.num_lanes
    NW = NC * NS
    assert (B * S) % NW == 0
    TPW = (B * S) // NW
    WPR = NW // B
    assert S % WPR == 0 and TPW == S // WPR
    CH = 16
    NBUF = 6
    DEPTH = 4
    NCHUNK = TPW // CH
    IPW = P // WPR

    mesh = plsc.VectorSubcoreMesh(core_axis_name="c", subcore_axis_name="s")

    @functools.partial(
        pl.kernel,
        out_type=[
            jax.ShapeDtypeStruct((B, S, H), jnp.float32),
            jax.ShapeDtypeStruct((B, S), jnp.int32),
        ],
        mesh=mesh,
        scratch_types=[
            pltpu.VMEM((S,), jnp.int32),
            pltpu.VMEM((L,), jnp.int32),
            pltpu.VMEM((TPW,), jnp.int32),
        ] + [pltpu.VMEM((CH, H), jnp.float32) for _ in range(NBUF)]
          + [pltpu.VMEM((L,), jnp.int32)]
          + [pltpu.SemaphoreType.DMA for _ in range(2 * NBUF + 1)],
    )
    def body(ids_hbm, img_hbm, tid_hbm, w_hbm, out_hbm, mask_hbm,
             row_v, tid_v, mask_v, *rest):
        bufs = rest[:NBUF]
        sidx_v = rest[NBUF]
        gsems = rest[NBUF + 1:2 * NBUF + 1]
        osems = rest[2 * NBUF + 1:3 * NBUF + 1]
        sem = rest[3 * NBUF + 1]
        c = lax.axis_index("c")
        s = lax.axis_index("s")
        wid = c * NS + s
        b = wid // WPR
        kw = wid % WPR
        loc = kw * TPW
        t0 = b * S + loc

        pltpu.sync_copy(ids_hbm.at[b], row_v)
        pltpu.sync_copy(tid_hbm, tid_v)
        tidv = tid_v[...]

        gd = [None] * NBUF
        od = [None] * NBUF

        def fire_g(k):
            gd[k % NBUF] = pltpu.async_copy(
                w_hbm.at[row_v.at[pl.ds(loc + k * CH, CH)]],
                bufs[k % NBUF], gsems[k % NBUF])

        def fire_o(k):
            od[k % NBUF] = pltpu.async_copy(
                bufs[k % NBUF], out_hbm.at[b, pl.ds(loc + k * CH, CH)],
                osems[k % NBUF])

        for k in range(DEPTH):
            fire_g(k)

        def scan_body(i, acc):
            v = row_v[pl.ds(i * L, L)]
            posv = lax.iota(jnp.int32, L) + i * L
            return jnp.minimum(acc, jnp.where(v == tidv, posv, S))

        fpv = lax.fori_loop(0, S // L, scan_body,
                            jnp.full((L,), S, jnp.int32))
        dnums = lax.GatherDimensionNumbers(
            offset_dims=(), collapsed_slice_dims=(0,), start_index_map=(0,))
        for st in (1, 2, 4, 8):
            perm = (lax.iota(jnp.int32, L) ^ st)[:, None]
            fpv = jnp.minimum(
                fpv, lax.gather(fpv, perm, dnums, (1,),
                                mode=lax.GatherScatterMode.PROMISE_IN_BOUNDS))
        fp = jnp.squeeze(lax.slice(fpv, (0,), (1,)))
        valid = fp <= S - P

        iota = lax.iota(jnp.int32, L)
        one = jnp.full((L,), 1, jnp.int32)
        zero = jnp.full((L,), 0, jnp.int32)
        for j in range(TPW // L):
            v = row_v[pl.ds(loc + j * L, L)]
            mask_v[pl.ds(j * L, L)] = jnp.where(v != -100, one, zero)
        pltpu.sync_copy(mask_v, mask_hbm.at[b, pl.ds(loc, TPW)])

        o_pending = [False] * NCHUNK
        for k in range(NCHUNK):
            gd[k % NBUF].wait()
            fire_o(k)
            o_pending[k] = True
            nk = k + DEPTH
            if nk < NCHUNK:
                j = nk - NBUF
                if j >= 0:
                    od[j % NBUF].wait()
                    o_pending[j] = False
                fire_g(nk)
        for k in range(NCHUNK):
            if o_pending[k]:
                od[k % NBUF].wait()

        plsc.subcore_barrier()

        @pl.when(valid)
        def _image_overwrite():
            for h in range(IPW // L):
                pltpu.sync_copy(
                    img_hbm.at[b, pl.ds(kw * IPW + h * L, L)], bufs[h % NBUF])
                sidx_v[...] = fpv + kw * IPW + h * L + iota
                pltpu.async_copy(bufs[h % NBUF].at[pl.ds(0, L)],
                                 out_hbm.at[b].at[sidx_v], sem).wait()

    return body


def kernel(input_ids, image_features, image_token_id, W):
    B, S = input_ids.shape
    _, P, H = image_features.shape
    V = W.shape[0]

    ids = input_ids.astype(jnp.int32)
    tid = jnp.full((16,), image_token_id, dtype=jnp.int32)

    sc = _build_sc_kernel(B, S, P, H, V)
    out, mask = sc(ids, image_features, tid, W)
    return out, mask


# speedup vs baseline: 4.1590x; 1.0103x over previous
import functools

import jax
import jax.numpy as jnp
from jax import lax
from jax.experimental import pallas as pl
from jax.experimental.pallas import tpu as pltpu
from jax.experimental.pallas import tpu_sc as plsc


def _build_sc_kernel(B, S, P, H, V):
    info = plsc.get_sparse_core_info()
    NC, NS, L = info.num_cores, info.num_subcores, info.num_lanes
    NW = NC * NS
    assert (B * S) % NW == 0
    TPW = (B * S) // NW
    WPR = NW // B
    assert S % WPR == 0 and TPW == S // WPR
    CH = 16
    NBUF = 6
    DEPTH = 5
    NCHUNK = TPW // CH
    IPW = P // WPR

    mesh = plsc.VectorSubcoreMesh(core_axis_name="c", subcore_axis_name="s")

    @functools.partial(
        pl.kernel,
        out_type=[
            jax.ShapeDtypeStruct((B, S, H), jnp.float32),
            jax.ShapeDtypeStruct((B, S), jnp.int32),
        ],
        mesh=mesh,
        scratch_types=[
            pltpu.VMEM((S,), jnp.int32),
            pltpu.VMEM((L,), jnp.int32),
            pltpu.VMEM((TPW,), jnp.int32),
        ] + [pltpu.VMEM((CH, H), jnp.float32) for _ in range(NBUF)]
          + [pltpu.VMEM((L,), jnp.int32)]
          + [pltpu.SemaphoreType.DMA for _ in range(2 * NBUF + 1)],
    )
    def body(ids_hbm, img_hbm, tid_hbm, w_hbm, out_hbm, mask_hbm,
             row_v, tid_v, mask_v, *rest):
        bufs = rest[:NBUF]
        sidx_v = rest[NBUF]
        gsems = rest[NBUF + 1:2 * NBUF + 1]
        osems = rest[2 * NBUF + 1:3 * NBUF + 1]
        sem = rest[3 * NBUF + 1]
        c = lax.axis_index("c")
        s = lax.axis_index("s")
        wid = c * NS + s
        b = wid // WPR
        kw = wid % WPR
        loc = kw * TPW
        t0 = b * S + loc

        pltpu.sync_copy(ids_hbm.at[b], row_v)
        pltpu.sync_copy(tid_hbm, tid_v)
        tidv = tid_v[...]

        gd = [None] * NBUF
        od = [None] * NBUF

        def fire_g(k):
            gd[k % NBUF] = pltpu.async_copy(
                w_hbm.at[row_v.at[pl.ds(loc + k * CH, CH)]],
                bufs[k % NBUF], gsems[k % NBUF])

        def fire_o(k):
            od[k % NBUF] = pltpu.async_copy(
                bufs[k % NBUF], out_hbm.at[b, pl.ds(loc + k * CH, CH)],
                osems[k % NBUF])

        for k in range(DEPTH):
            fire_g(k)

        def scan_body(i, acc):
            v = row_v[pl.ds(i * L, L)]
            posv = lax.iota(jnp.int32, L) + i * L
            return jnp.minimum(acc, jnp.where(v == tidv, posv, S))

        fpv = lax.fori_loop(0, S // L, scan_body,
                            jnp.full((L,), S, jnp.int32))
        dnums = lax.GatherDimensionNumbers(
            offset_dims=(), collapsed_slice_dims=(0,), start_index_map=(0,))
        for st in (1, 2, 4, 8):
            perm = (lax.iota(jnp.int32, L) ^ st)[:, None]
            fpv = jnp.minimum(
                fpv, lax.gather(fpv, perm, dnums, (1,),
                                mode=lax.GatherScatterMode.PROMISE_IN_BOUNDS))
        fp = jnp.squeeze(lax.slice(fpv, (0,), (1,)))
        valid = fp <= S - P

        iota = lax.iota(jnp.int32, L)
        one = jnp.full((L,), 1, jnp.int32)
        zero = jnp.full((L,), 0, jnp.int32)
        for j in range(TPW // L):
            v = row_v[pl.ds(loc + j * L, L)]
            mask_v[pl.ds(j * L, L)] = jnp.where(v != -100, one, zero)
        pltpu.sync_copy(mask_v, mask_hbm.at[b, pl.ds(loc, TPW)])

        o_pending = [False] * NCHUNK
        for k in range(NCHUNK):
            gd[k % NBUF].wait()
            fire_o(k)
            o_pending[k] = True
            nk = k + DEPTH
            if nk < NCHUNK:
                j = nk - NBUF
                if j >= 0:
                    od[j % NBUF].wait()
                    o_pending[j] = False
                fire_g(nk)
        for k in range(NCHUNK):
            if o_pending[k]:
                od[k % NBUF].wait()

        plsc.subcore_barrier()

        @pl.when(valid)
        def _image_overwrite():
            for h in range(IPW // L):
                pltpu.sync_copy(
                    img_hbm.at[b, pl.ds(kw * IPW + h * L, L)], bufs[h % NBUF])
                sidx_v[...] = fpv + kw * IPW + h * L + iota
                pltpu.async_copy(bufs[h % NBUF].at[pl.ds(0, L)],
                                 out_hbm.at[b].at[sidx_v], sem).wait()

    return body


def kernel(input_ids, image_features, image_token_id, W):
    B, S = input_ids.shape
    _, P, H = image_features.shape
    V = W.shape[0]

    ids = input_ids.astype(jnp.int32)
    tid = jnp.full((16,), image_token_id, dtype=jnp.int32)

    sc = _build_sc_kernel(B, S, P, H, V)
    out, mask = sc(ids, image_features, tid, W)
    return out, mask
